# baseline (speedup 1.0000x reference)
import functools

import jax
import jax.numpy as jnp
from jax import lax
from jax.experimental import pallas as pl
from jax.experimental.pallas import tpu as pltpu
from jax.experimental.pallas import tpu_sc as plsc

B = 16384
EMB = 64
NC = 2
NS = 16
NW = NC * NS
BPW = B // NW
CHUNK = 64
NCHUNK = BPW // CHUNK
NT = 7


def _sc_kernel(r_idx, e1_idx, e2_idx, e3_idx, e4_idx, e5_idx, e6_idx,
               E1, E2, E3, E4, E5, E6, R, out, *scratch):
    idx_hbm = (r_idx, e1_idx, e2_idx, e3_idx, e4_idx, e5_idx, e6_idx)
    tables = (R, E1, E2, E3, E4, E5, E6)
    idx_v = scratch[0:NT]
    rows_v = scratch[NT:NT + 2 * NT]
    out_v = scratch[2 * NT + NT]
    sems = scratch[2 * NT + NT + 1:]

    wid = lax.axis_index("s") * NC + lax.axis_index("c")
    base = wid * BPW

    for t in range(NT):
        pltpu.sync_copy(idx_hbm[t].at[pl.ds(base, BPW)], idx_v[t])

    def start_gathers(c, slot):
        cps = []
        for t in range(NT):
            cps.append(pltpu.async_copy(
                tables[t].at[idx_v[t].at[pl.ds(c * CHUNK, CHUNK)]],
                rows_v[slot * NT + t], sems[slot]))
        return cps

    def compute(c, slot):
        bufs = rows_v[slot * NT:slot * NT + NT]
        for g in range(CHUNK // 16):
            row_ids = lax.iota(jnp.int32, 16) + (g * 16)

            def body(d, acc):
                col = jnp.full((16,), d, jnp.int32)
                p = plsc.load_gather(bufs[0], [row_ids, col])
                for t in range(1, NT):
                    p = p * plsc.load_gather(bufs[t], [row_ids, col])
                return acc + p

            acc = lax.fori_loop(0, EMB, body, jnp.zeros((16,), jnp.float32))
            out_v[pl.ds(c * CHUNK + g * 16, 16)] = acc

    pending = start_gathers(0, 0)
    for c in range(NCHUNK):
        nxt = None
        if c + 1 < NCHUNK:
            nxt = start_gathers(c + 1, (c + 1) % 2)
        for cp in pending:
            cp.wait()
        compute(c, c % 2)
        pending = nxt

    pltpu.sync_copy(out_v, out.at[pl.ds(base, BPW)])


def kernel(r_idx, e1_idx, e2_idx, e3_idx, e4_idx, e5_idx, e6_idx,
           E1, E2, E3, E4, E5, E6, R):
    idxs = [jnp.asarray(i, jnp.int32)
            for i in (r_idx, e1_idx, e2_idx, e3_idx, e4_idx, e5_idx, e6_idx)]
    mesh = plsc.VectorSubcoreMesh(core_axis_name="c", subcore_axis_name="s")
    scratch = (
        [pltpu.VMEM((BPW,), jnp.int32) for _ in range(NT)]
        + [pltpu.VMEM((CHUNK, EMB), jnp.float32) for _ in range(2 * NT)]
        + [pltpu.VMEM((BPW,), jnp.float32)]
        + [pltpu.SemaphoreType.DMA, pltpu.SemaphoreType.DMA]
    )
    f = functools.partial(
        pl.kernel, mesh=mesh,
        out_type=jax.ShapeDtypeStruct((B,), jnp.float32),
        scratch_types=scratch,
        compiler_params=pltpu.CompilerParams(
            needs_layout_passes=False, use_tc_tiling_on_sc=False),
    )(_sc_kernel)
    return f(*idxs, E1, E2, E3, E4, E5, E6, R)


# speedup vs baseline: 1.4022x; 1.4022x over previous
import functools

import jax
import jax.numpy as jnp
from jax import lax
from jax.experimental import pallas as pl
from jax.experimental.pallas import tpu as pltpu
from jax.experimental.pallas import tpu_sc as plsc

B = 16384
EMB = 64
W = 128
NC = 2
NS = 16
NW = NC * NS
BPW = B // NW
CHUNK = 64
NCHUNK = BPW // CHUNK
NT = 7
PAIR = (0, 1, 1, 2, 2, 3, 3)
HALF = (0, 0, 1, 0, 1, 0, 1)


def _sc_kernel(r_idx, e1_idx, e2_idx, e3_idx, e4_idx, e5_idx, e6_idx,
               F0, F1, F2, F3, out, *scratch):
    idx_hbm = (r_idx, e1_idx, e2_idx, e3_idx, e4_idx, e5_idx, e6_idx)
    tables = (F0, F1, F2, F3)
    idx_v = scratch[0:NT]
    rows_v = scratch[NT:NT + 2 * NT]
    out_v = scratch[2 * NT + NT]
    tr_v = scratch[2 * NT + NT + 1]
    sems = scratch[2 * NT + NT + 2:]

    wid = lax.axis_index("s") * NC + lax.axis_index("c")
    base = wid * BPW

    for t in range(NT):
        pltpu.sync_copy(idx_hbm[t].at[pl.ds(base, BPW)], idx_v[t])

    def gathers(c, slot):
        return [pltpu.make_async_copy(
            tables[PAIR[t]].at[idx_v[t].at[pl.ds(c * CHUNK, CHUNK)]],
            rows_v[slot * NT + t], sems[slot]) for t in range(NT)]

    def start(c, slot):
        for cp in gathers(c, slot):
            cp.start()

    def wait(c, slot):
        for cp in gathers(c, slot):
            cp.wait()

    def compute(c, slot):
        bufs = rows_v[slot * NT:slot * NT + NT]

        def group(g, _):
            rbase = c * CHUNK + g * 16
            for r in range(16):
                s = None
                for q in range(4):
                    p = bufs[0][g * 16 + r, pl.ds(HALF[0] * EMB + q * 16, 16)]
                    for t in range(1, NT):
                        p = p * bufs[t][g * 16 + r,
                                        pl.ds(HALF[t] * EMB + q * 16, 16)]
                    s = p if s is None else s + p
                tr_v[pl.ds(r * 17, 16)] = s
            acc = jnp.zeros((16,), jnp.float32)
            cols = lax.iota(jnp.int32, 16) * 17
            for l in range(16):
                acc = acc + plsc.load_gather(tr_v, [cols + l])
            out_v[pl.ds(rbase, 16)] = acc
            return 0

        lax.fori_loop(0, CHUNK // 16, group, 0)

    start(0, 0)
    start(1, 1)

    def chunk_pair(cp_i, _):
        c = cp_i * 2
        wait(c, 0)
        compute(c, 0)

        @pl.when(cp_i + 1 < NCHUNK // 2)
        def _():
            start(c + 2, 0)
        wait(c + 1, 1)
        compute(c + 1, 1)

        @pl.when(cp_i + 1 < NCHUNK // 2)
        def _():
            start(c + 3, 1)
        return 0

    lax.fori_loop(0, NCHUNK // 2, chunk_pair, 0)

    pltpu.sync_copy(out_v, out.at[pl.ds(base, BPW)])


def kernel(r_idx, e1_idx, e2_idx, e3_idx, e4_idx, e5_idx, e6_idx,
           E1, E2, E3, E4, E5, E6, R):
    idxs = [jnp.asarray(i, jnp.int32)
            for i in (r_idx, e1_idx, e2_idx, e3_idx, e4_idx, e5_idx, e6_idx)]
    F0 = jnp.concatenate([R, R], axis=1)
    F1 = jnp.concatenate([E1, E2], axis=1)
    F2 = jnp.concatenate([E3, E4], axis=1)
    F3 = jnp.concatenate([E5, E6], axis=1)
    mesh = plsc.VectorSubcoreMesh(core_axis_name="c", subcore_axis_name="s")
    scratch = (
        [pltpu.VMEM((BPW,), jnp.int32) for _ in range(NT)]
        + [pltpu.VMEM((CHUNK, W), jnp.float32) for _ in range(2 * NT)]
        + [pltpu.VMEM((BPW,), jnp.float32)]
        + [pltpu.VMEM((16 * 17,), jnp.float32)]
        + [pltpu.SemaphoreType.DMA, pltpu.SemaphoreType.DMA]
    )
    f = functools.partial(
        pl.kernel, mesh=mesh,
        out_type=jax.ShapeDtypeStruct((B,), jnp.float32),
        scratch_types=scratch,
        compiler_params=pltpu.CompilerParams(
            needs_layout_passes=False, use_tc_tiling_on_sc=True),
    )(_sc_kernel)
    return f(*idxs, F0, F1, F2, F3)
